# baseline (speedup 1.0000x reference)
import jax
import jax.numpy as jnp
from jax import lax
from jax.experimental import pallas as pl
from jax.experimental.pallas import tpu as pltpu
from jax.experimental.pallas import tpu_sc as plsc

N_OUT = 4096
N_IN = 4096
K = 32
E = 8
D_VAL = 128
P_PAD = 16
ROW_BLK = 256
N_BLKS = N_OUT // ROW_BLK

_NW = 32
_BPW = (N_OUT * K) // _NW
_CHUNK = 128
_NCH = _BPW // _CHUNK


def _topk_body(po_ref, pi_ref, idx_ref, d_ref):
    dx = po_ref[:, 0:1] - pi_ref[0:1, :]
    dy = po_ref[:, 1:2] - pi_ref[1:2, :]
    dz = po_ref[:, 2:3] - pi_ref[2:3, :]
    d0 = dx * dx + dy * dy + dz * dz
    d_ref[...] = d0
    col = lax.broadcasted_iota(jnp.int32, (ROW_BLK, N_IN), 1)
    kcol = lax.broadcasted_iota(jnp.int32, (ROW_BLK, K), 1)
    idx0 = jnp.argmin(d0, axis=1).astype(jnp.int32)[:, None]

    def step(k, carry):
        acc, idxv = carry
        dn = jnp.where(col == idxv, jnp.float32(jnp.inf), d_ref[...])
        d_ref[...] = dn
        nidx = jnp.argmin(dn, axis=1).astype(jnp.int32)[:, None]
        return jnp.where(kcol == k, idxv, acc), nidx

    acc, _ = lax.fori_loop(0, K, step,
                           (jnp.zeros((ROW_BLK, K), jnp.int32), idx0))
    idx_ref[...] = acc


def _topk(points_out, points_in_t):
    return pl.pallas_call(
        _topk_body,
        grid=(N_BLKS,),
        in_specs=[
            pl.BlockSpec((ROW_BLK, 3), lambda i: (i, 0)),
            pl.BlockSpec((3, N_IN), lambda i: (0, 0)),
        ],
        out_specs=pl.BlockSpec((ROW_BLK, K), lambda i: (i, 0)),
        out_shape=jax.ShapeDtypeStruct((N_OUT, K), jnp.int32),
        scratch_shapes=[pltpu.VMEM((ROW_BLK, N_IN), jnp.float32)],
    )(points_out, points_in_t)


def _sc_gather_body(values_hbm, points_hbm, idx2d_hbm,
                    vals_out, pts_out,
                    idx_v, rows_v, prow_v, sem):
    wid = lax.axis_index("s") * 2 + lax.axis_index("c")
    base = wid * _BPW
    pltpu.sync_copy(idx2d_hbm.at[pl.ds(wid * _NCH, _NCH)], idx_v)

    def gbody(c, carry):
        pltpu.async_copy(values_hbm.at[idx_v.at[c]], rows_v, sem).wait()
        pltpu.sync_copy(rows_v,
                        vals_out.at[pl.ds(base + c * _CHUNK, _CHUNK)])
        pltpu.async_copy(points_hbm.at[idx_v.at[c]], prow_v, sem).wait()
        pltpu.sync_copy(prow_v,
                        pts_out.at[pl.ds(base + c * _CHUNK, _CHUNK)])
        return carry

    lax.fori_loop(0, _NCH, gbody, 0)


def _sc_gather(values_in, points_pad, idx2d):
    mesh = plsc.VectorSubcoreMesh(core_axis_name="c", subcore_axis_name="s")
    fn = pl.kernel(
        _sc_gather_body,
        out_type=(
            jax.ShapeDtypeStruct((N_OUT * K, D_VAL), jnp.float32),
            jax.ShapeDtypeStruct((N_OUT * K, P_PAD), jnp.float32),
        ),
        mesh=mesh,
        compiler_params=pltpu.CompilerParams(use_tc_tiling_on_sc=False),
        scratch_types=[
            pltpu.VMEM((_NCH, _CHUNK), jnp.int32),
            pltpu.VMEM((_CHUNK, D_VAL), jnp.float32),
            pltpu.VMEM((_CHUNK, P_PAD), jnp.float32),
            pltpu.SemaphoreType.DMA,
        ],
    )
    return fn(values_in, points_pad, idx2d)


def _conv_body(po_ref, gp_ref, vals_ref, cen_ref, coefft_ref, bias_ref,
               out_ref, ux_ref, uy_ref, uz_ref):
    ux = po_ref[:, 0:1] - gp_ref[:, :, 0]
    uy = po_ref[:, 1:2] - gp_ref[:, :, 1]
    uz = po_ref[:, 2:3] - gp_ref[:, :, 2]
    ux_ref[...] = ux
    uy_ref[...] = uy
    uz_ref[...] = uz
    vals = vals_ref[...]
    acc = jnp.zeros((ROW_BLK, D_VAL), jnp.float32)
    for e in range(E):
        cx = cen_ref[0:1, e:e + 1]
        cy = cen_ref[1:2, e:e + 1]
        cz = cen_ref[2:3, e:e + 1]
        kb = jnp.exp(-((ux - cx) ** 2 + (uy - cy) ** 2 + (uz - cz) ** 2))
        tmp = jnp.sum(kb[:, :, None] * vals, axis=1)
        acc = acc + jnp.dot(tmp, coefft_ref[e],
                            preferred_element_type=jnp.float32)
    out_ref[...] = acc * jnp.float32(1.0 / K) + bias_ref[...]


def _conv(po, gp3, vals_t, cen_t, coefft, bias2d):
    return pl.pallas_call(
        _conv_body,
        grid=(N_BLKS,),
        in_specs=[
            pl.BlockSpec((ROW_BLK, 3), lambda i: (i, 0)),
            pl.BlockSpec((ROW_BLK, K, P_PAD), lambda i: (i, 0, 0)),
            pl.BlockSpec((ROW_BLK, K, D_VAL), lambda i: (i, 0, 0)),
            pl.BlockSpec((3, E), lambda i: (0, 0)),
            pl.BlockSpec((E, D_VAL, D_VAL), lambda i: (0, 0, 0)),
            pl.BlockSpec((1, D_VAL), lambda i: (0, 0)),
        ],
        out_specs=[
            pl.BlockSpec((ROW_BLK, D_VAL), lambda i: (i, 0)),
            pl.BlockSpec((ROW_BLK, K), lambda i: (i, 0)),
            pl.BlockSpec((ROW_BLK, K), lambda i: (i, 0)),
            pl.BlockSpec((ROW_BLK, K), lambda i: (i, 0)),
        ],
        out_shape=[
            jax.ShapeDtypeStruct((N_OUT, D_VAL), jnp.float32),
            jax.ShapeDtypeStruct((N_OUT, K), jnp.float32),
            jax.ShapeDtypeStruct((N_OUT, K), jnp.float32),
            jax.ShapeDtypeStruct((N_OUT, K), jnp.float32),
        ],
    )(po, gp3, vals_t, cen_t, coefft, bias2d)


def kernel(points_in, values_in, points_out, coeff, bias, rbf_centers):
    pi_t = points_in.T
    idx = _topk(points_out, pi_t)
    idx2d = idx.reshape(-1, _CHUNK)
    points_pad = jnp.pad(points_in, ((0, 0), (0, P_PAD - 3)))
    vals_k, pts_k = _sc_gather(values_in, points_pad, idx2d)
    out, ux, uy, uz = _conv(
        points_out,
        pts_k.reshape(N_OUT, K, P_PAD),
        vals_k.reshape(N_OUT, K, D_VAL),
        rbf_centers.T, jnp.transpose(coeff, (2, 0, 1)),
        bias.reshape(1, D_VAL))
    uiv = jnp.stack([ux, uy, uz], axis=-1)
    return (uiv, idx.astype(jnp.int64), out)


# speedup vs baseline: 1.2188x; 1.2188x over previous
import jax
import jax.numpy as jnp
from jax import lax
from jax.experimental import pallas as pl
from jax.experimental.pallas import tpu as pltpu
from jax.experimental.pallas import tpu_sc as plsc

N_OUT = 4096
N_IN = 4096
K = 32
E = 8
D_VAL = 128
P_PAD = 16
ROW_BLK = 256
N_BLKS = N_OUT // ROW_BLK

_NW = 32
_BPW = (N_OUT * K) // _NW
_CHUNK = 128
_NCH = _BPW // _CHUNK


def _topk_body(po_ref, pi_ref, idx_ref, d_ref):
    dx = po_ref[:, 0:1] - pi_ref[0:1, :]
    dy = po_ref[:, 1:2] - pi_ref[1:2, :]
    dz = po_ref[:, 2:3] - pi_ref[2:3, :]
    d_ref[...] = dx * dx + dy * dy + dz * dz
    col = lax.broadcasted_iota(jnp.int32, (ROW_BLK, N_IN), 1)
    kcol = lax.broadcasted_iota(jnp.int32, (ROW_BLK, K), 1)

    def step(k, acc):
        idxv = jnp.argmin(d_ref[...], axis=1).astype(jnp.int32)[:, None]
        d_ref[...] = jnp.where(col == idxv, jnp.float32(jnp.inf), d_ref[...])
        return jnp.where(kcol == k, idxv, acc)

    idx_ref[...] = lax.fori_loop(0, K, step,
                                 jnp.zeros((ROW_BLK, K), jnp.int32))


def _topk(points_out, points_in_t):
    return pl.pallas_call(
        _topk_body,
        grid=(N_BLKS,),
        in_specs=[
            pl.BlockSpec((ROW_BLK, 3), lambda i: (i, 0)),
            pl.BlockSpec((3, N_IN), lambda i: (0, 0)),
        ],
        out_specs=pl.BlockSpec((ROW_BLK, K), lambda i: (i, 0)),
        out_shape=jax.ShapeDtypeStruct((N_OUT, K), jnp.int32),
        scratch_shapes=[pltpu.VMEM((ROW_BLK, N_IN), jnp.float32)],
    )(points_out, points_in_t)


def _sc_gather_body(values_hbm, points_hbm, idx2d_hbm,
                    vals_out, pts_out,
                    idx_v, rows_v, prow_v, sem):
    wid = lax.axis_index("s") * 2 + lax.axis_index("c")
    base = wid * _BPW
    pltpu.sync_copy(idx2d_hbm.at[pl.ds(wid * _NCH, _NCH)], idx_v)

    def gbody(c, carry):
        pltpu.async_copy(values_hbm.at[idx_v.at[c]], rows_v, sem).wait()
        pltpu.sync_copy(rows_v,
                        vals_out.at[pl.ds(base + c * _CHUNK, _CHUNK)])
        pltpu.async_copy(points_hbm.at[idx_v.at[c]], prow_v, sem).wait()
        pltpu.sync_copy(prow_v,
                        pts_out.at[pl.ds(base + c * _CHUNK, _CHUNK)])
        return carry

    lax.fori_loop(0, _NCH, gbody, 0)


def _sc_gather(values_in, points_pad, idx2d):
    mesh = plsc.VectorSubcoreMesh(core_axis_name="c", subcore_axis_name="s")
    fn = pl.kernel(
        _sc_gather_body,
        out_type=(
            jax.ShapeDtypeStruct((N_OUT * K, D_VAL), jnp.float32),
            jax.ShapeDtypeStruct((N_OUT * K, P_PAD), jnp.float32),
        ),
        mesh=mesh,
        compiler_params=pltpu.CompilerParams(use_tc_tiling_on_sc=False),
        scratch_types=[
            pltpu.VMEM((_NCH, _CHUNK), jnp.int32),
            pltpu.VMEM((_CHUNK, D_VAL), jnp.float32),
            pltpu.VMEM((_CHUNK, P_PAD), jnp.float32),
            pltpu.SemaphoreType.DMA,
        ],
    )
    return fn(values_in, points_pad, idx2d)


def _conv_body(po_ref, gp_ref, vals_ref, cen_ref, coefft_ref, bias_ref,
               out_ref, ux_ref, uy_ref, uz_ref):
    ux = po_ref[:, 0:1] - gp_ref[:, :, 0]
    uy = po_ref[:, 1:2] - gp_ref[:, :, 1]
    uz = po_ref[:, 2:3] - gp_ref[:, :, 2]
    ux_ref[...] = ux
    uy_ref[...] = uy
    uz_ref[...] = uz
    vals = vals_ref[...]
    kbs = []
    for e in range(E):
        cx = cen_ref[0:1, e:e + 1]
        cy = cen_ref[1:2, e:e + 1]
        cz = cen_ref[2:3, e:e + 1]
        kbs.append(jnp.exp(-((ux - cx) ** 2 + (uy - cy) ** 2
                             + (uz - cz) ** 2)))
    kb_all = jnp.stack(kbs, axis=1)
    tmp_all = jax.lax.dot_general(
        kb_all, vals,
        dimension_numbers=(((2,), (1,)), ((0,), (0,))),
        preferred_element_type=jnp.float32)
    acc = jnp.zeros((ROW_BLK, D_VAL), jnp.float32)
    for e in range(E):
        acc = acc + jnp.dot(tmp_all[:, e, :], coefft_ref[e],
                            preferred_element_type=jnp.float32)
    out_ref[...] = acc * jnp.float32(1.0 / K) + bias_ref[...]


def _conv(po, gp3, vals_t, cen_t, coefft, bias2d):
    return pl.pallas_call(
        _conv_body,
        grid=(N_BLKS,),
        in_specs=[
            pl.BlockSpec((ROW_BLK, 3), lambda i: (i, 0)),
            pl.BlockSpec((ROW_BLK, K, P_PAD), lambda i: (i, 0, 0)),
            pl.BlockSpec((ROW_BLK, K, D_VAL), lambda i: (i, 0, 0)),
            pl.BlockSpec((3, E), lambda i: (0, 0)),
            pl.BlockSpec((E, D_VAL, D_VAL), lambda i: (0, 0, 0)),
            pl.BlockSpec((1, D_VAL), lambda i: (0, 0)),
        ],
        out_specs=[
            pl.BlockSpec((ROW_BLK, D_VAL), lambda i: (i, 0)),
            pl.BlockSpec((ROW_BLK, K), lambda i: (i, 0)),
            pl.BlockSpec((ROW_BLK, K), lambda i: (i, 0)),
            pl.BlockSpec((ROW_BLK, K), lambda i: (i, 0)),
        ],
        out_shape=[
            jax.ShapeDtypeStruct((N_OUT, D_VAL), jnp.float32),
            jax.ShapeDtypeStruct((N_OUT, K), jnp.float32),
            jax.ShapeDtypeStruct((N_OUT, K), jnp.float32),
            jax.ShapeDtypeStruct((N_OUT, K), jnp.float32),
        ],
    )(po, gp3, vals_t, cen_t, coefft, bias2d)


def kernel(points_in, values_in, points_out, coeff, bias, rbf_centers):
    pi_t = points_in.T
    idx = _topk(points_out, pi_t)
    idx2d = idx.reshape(-1, _CHUNK)
    points_pad = jnp.pad(points_in, ((0, 0), (0, P_PAD - 3)))
    vals_k, pts_k = _sc_gather(values_in, points_pad, idx2d)
    out, ux, uy, uz = _conv(
        points_out,
        pts_k.reshape(N_OUT, K, P_PAD),
        vals_k.reshape(N_OUT, K, D_VAL),
        rbf_centers.T, jnp.transpose(coeff, (2, 0, 1)),
        bias.reshape(1, D_VAL))
    uiv = jnp.stack([ux, uy, uz], axis=-1)
    return (uiv, idx.astype(jnp.int64), out)
